# baseline (speedup 1.0000x reference)
import jax
import jax.numpy as jnp
from jax import lax
from jax.experimental import pallas as pl
from jax.experimental.pallas import tpu as pltpu
from jax.experimental.pallas import tpu_sc as plsc

V = 32000
PAD = 0
N = 1600
NC = 2
NS = 16
L = 16
NW = NC * NS
PER_W = 64
NPAD = NW * PER_W
CHUNKS = PER_W // L


def _loss_body(scores_hbm, tgt_hbm, out_hbm, tgt_v, idx_v, val_v, red_v,
               all_v, shared, sem):
    cid = lax.axis_index("c")
    sid = lax.axis_index("s")
    wid = cid * NS + sid
    base = wid * PER_W

    pltpu.sync_copy(tgt_hbm.at[pl.ds(base, PER_W)], tgt_v)

    lane = lax.iota(jnp.int32, L)
    for j in range(CHUNKS):
        t = tgt_v[pl.ds(j * L, L)]
        r = base + j * L + lane
        r = jnp.where(r < N, r, r - N)
        idx_v[pl.ds(j * L, L)] = r * V + t

    pltpu.async_copy(scores_hbm.at[idx_v], val_v, sem).wait()

    acc = jnp.zeros((L,), jnp.float32)
    for j in range(CHUNKS):
        t = tgt_v[pl.ds(j * L, L)]
        v = val_v[pl.ds(j * L, L)]
        acc = acc + jnp.where(t != PAD, v, 0.0)

    red_v[...] = acc
    pltpu.sync_copy(red_v, shared.at[pl.ds(sid * L, L)])
    plsc.subcore_barrier()

    @pl.when(sid == 0)
    def _():
        pltpu.sync_copy(shared, all_v)
        tot = all_v[pl.ds(0, L)]
        for i in range(1, NS):
            tot = tot + all_v[pl.ds(i * L, L)]
        red_v[...] = tot
        pltpu.sync_copy(red_v, out_hbm.at[cid])


def kernel(outputs, targets):
    scores = outputs.reshape(-1)
    tgt = jnp.ravel(targets).astype(jnp.int32)
    tgt = jnp.concatenate([tgt, jnp.zeros((NPAD - N,), jnp.int32)])

    mesh = plsc.VectorSubcoreMesh(
        core_axis_name="c", subcore_axis_name="s",
        num_cores=NC, num_subcores=NS)
    run = pl.kernel(
        _loss_body,
        out_type=jax.ShapeDtypeStruct((NC, L), jnp.float32),
        mesh=mesh,
        scratch_types=[
            pltpu.VMEM((PER_W,), jnp.int32),
            pltpu.VMEM((PER_W,), jnp.int32),
            pltpu.VMEM((PER_W,), jnp.float32),
            pltpu.VMEM((L,), jnp.float32),
            pltpu.VMEM((NS * L,), jnp.float32),
            pltpu.VMEM_SHARED((NS * L,), jnp.float32),
            pltpu.SemaphoreType.DMA,
        ],
    )
    out = run(scores, tgt)
    return -jnp.sum(out)


# speedup vs baseline: 6.1352x; 6.1352x over previous
import jax
import jax.numpy as jnp
from jax import lax
from jax.experimental import pallas as pl
from jax.experimental.pallas import tpu as pltpu
from jax.experimental.pallas import tpu_sc as plsc

V = 32000
PAD = 0
N = 1600
NC = 2
NS = 16
L = 16
NW = NC * NS
PER_W = N // NW
SLOT = 64
CHUNKS = 4


def _loss_body(scores_hbm, tgt_hbm, out_hbm, tgt_v, buf_v, red_v, all_v,
               shared, sem):
    cid = lax.axis_index("c")
    sid = lax.axis_index("s")
    wid = cid * NS + sid
    base = wid * PER_W

    pltpu.sync_copy(tgt_hbm.at[pl.ds(wid * SLOT, SLOT)], tgt_v)
    chunks = [tgt_v[pl.ds(j * L, L)] for j in range(CHUNKS)]

    copies = []
    for i in range(PER_W):
        t = chunks[i // L][i % L]
        r0 = pl.multiple_of(((base + i) // 8) * 8, 8)
        c0 = pl.multiple_of((t >> 7) << 7, 128)
        copies.append(pltpu.make_async_copy(
            scores_hbm.at[pl.ds(r0, 8), pl.ds(c0, 128)], buf_v.at[i], sem))
    for c in copies:
        c.start()
    for c in copies:
        c.wait()

    lanes = lax.iota(jnp.int32, L)
    acc = jnp.zeros((L,), jnp.float32)
    for j in range(CHUNKS):
        k = j * L
        t = chunks[j]
        valid = (k + lanes) < PER_W
        rows = jnp.where(valid, k + lanes, 0)
        sub = jnp.where(valid, (base + k + lanes) & 7, 0)
        lane = jnp.where(valid, t & 127, 0)
        vals = plsc.load_gather(buf_v, [rows, sub, lane])
        acc = acc + jnp.where(valid & (t != PAD), vals, 0.0)

    red_v[...] = acc
    pltpu.sync_copy(red_v, shared.at[pl.ds(sid * L, L)])
    plsc.subcore_barrier()

    @pl.when(sid == 0)
    def _():
        pltpu.sync_copy(shared, all_v)
        tot = all_v[pl.ds(0, L)]
        for i in range(1, NS):
            tot = tot + all_v[pl.ds(i * L, L)]
        red_v[...] = tot
        pltpu.sync_copy(red_v, out_hbm.at[cid])


def kernel(outputs, targets):
    scores = outputs.reshape(N, V)
    tgt = jnp.ravel(targets).astype(jnp.int32)
    tgt = jnp.pad(tgt.reshape(NW, PER_W),
                  ((0, 0), (0, SLOT - PER_W))).reshape(-1)

    mesh = plsc.VectorSubcoreMesh(
        core_axis_name="c", subcore_axis_name="s",
        num_cores=NC, num_subcores=NS)
    run = pl.kernel(
        _loss_body,
        out_type=jax.ShapeDtypeStruct((NC, L), jnp.float32),
        mesh=mesh,
        compiler_params=pltpu.CompilerParams(use_tc_tiling_on_sc=True,
                                             needs_layout_passes=False),
        scratch_types=[
            pltpu.VMEM((SLOT,), jnp.int32),
            pltpu.VMEM((PER_W, 8, 128), jnp.float32),
            pltpu.VMEM((L,), jnp.float32),
            pltpu.VMEM((NS * L,), jnp.float32),
            pltpu.VMEM_SHARED((NS * L,), jnp.float32),
            pltpu.SemaphoreType.DMA,
        ],
    )
    out = run(scores, tgt)
    return -jnp.sum(out)


# speedup vs baseline: 6.6689x; 1.0870x over previous
import jax
import jax.numpy as jnp
from jax import lax
from jax.experimental import pallas as pl
from jax.experimental.pallas import tpu as pltpu
from jax.experimental.pallas import tpu_sc as plsc

V = 32000
PAD = 0
N = 1600
NC = 2
NS = 16
L = 16
NW = NC * NS
PER_W = N // NW
SLOT = 64
CHUNKS = SLOT // L


def _loss_body(table_hbm, tgt_hbm, out_hbm, tgt_v, idx_v, buf_v, red_v,
               all_v, shared, sem):
    cid = lax.axis_index("c")
    sid = lax.axis_index("s")
    wid = cid * NS + sid
    base = wid * PER_W

    pltpu.sync_copy(tgt_hbm.at[pl.ds(wid * SLOT, SLOT)], tgt_v)

    lanes = lax.iota(jnp.int32, L)
    for j in range(CHUNKS):
        k = j * L
        t = tgt_v[pl.ds(k, L)]
        valid = (k + lanes) < PER_W
        r = base + k + lanes
        p = ((r >> 3) * 2000) + ((t >> 7) << 3) + (r & 7)
        idx_v[pl.ds(k, L)] = jnp.where(valid, p, k + lanes)

    pltpu.async_copy(table_hbm.at[idx_v], buf_v, sem).wait()

    acc = jnp.zeros((L,), jnp.float32)
    for j in range(CHUNKS):
        k = j * L
        t = tgt_v[pl.ds(k, L)]
        valid = (k + lanes) < PER_W
        rows = k + lanes
        lane = jnp.where(valid, t & 127, 0)
        vals = plsc.load_gather(buf_v, [rows, lane])
        acc = acc + jnp.where(valid & (t != PAD), vals, 0.0)

    red_v[...] = acc
    pltpu.sync_copy(red_v, shared.at[pl.ds(sid * L, L)])
    plsc.subcore_barrier()

    @pl.when(sid == 0)
    def _():
        pltpu.sync_copy(shared, all_v)
        tot = all_v[pl.ds(0, L)]
        for i in range(1, NS):
            tot = tot + all_v[pl.ds(i * L, L)]
        red_v[...] = tot
        pltpu.sync_copy(red_v, out_hbm.at[cid])


def kernel(outputs, targets):
    table = outputs.reshape(200, 8, 250, 128).transpose(0, 2, 1, 3)
    table = table.reshape(N * V // 128, 128)
    tgt = jnp.ravel(targets).astype(jnp.int32)
    tgt = jnp.pad(tgt.reshape(NW, PER_W),
                  ((0, 0), (0, SLOT - PER_W))).reshape(-1)

    mesh = plsc.VectorSubcoreMesh(
        core_axis_name="c", subcore_axis_name="s",
        num_cores=NC, num_subcores=NS)
    run = pl.kernel(
        _loss_body,
        out_type=jax.ShapeDtypeStruct((NC, L), jnp.float32),
        mesh=mesh,
        compiler_params=pltpu.CompilerParams(use_tc_tiling_on_sc=True,
                                             needs_layout_passes=False),
        scratch_types=[
            pltpu.VMEM((SLOT,), jnp.int32),
            pltpu.VMEM((SLOT,), jnp.int32),
            pltpu.VMEM((SLOT, 128), jnp.float32),
            pltpu.VMEM((L,), jnp.float32),
            pltpu.VMEM((NS * L,), jnp.float32),
            pltpu.VMEM_SHARED((NS * L,), jnp.float32),
            pltpu.SemaphoreType.DMA,
        ],
    )
    out = run(table, tgt)
    return -jnp.sum(out)


# speedup vs baseline: 7.1646x; 1.0743x over previous
import jax
import jax.numpy as jnp
from jax import lax
from jax.experimental import pallas as pl
from jax.experimental.pallas import tpu as pltpu
from jax.experimental.pallas import tpu_sc as plsc

V = 32000
PAD = 0
N = 1600
NC = 2
NS = 16
L = 16
NW = NC * NS
PER_W = N // NW
SLOT = 64
CHUNKS = SLOT // L


def _loss_body(table_hbm, tgt_hbm, out_hbm, tgt_v, idx_v, red_v, buf_v, sem):
    cid = lax.axis_index("c")
    sid = lax.axis_index("s")
    wid = cid * NS + sid
    base = wid * PER_W

    pltpu.sync_copy(tgt_hbm.at[pl.ds(wid * SLOT, SLOT)], tgt_v)

    lanes = lax.iota(jnp.int32, L)
    for j in range(CHUNKS):
        k = j * L
        t = tgt_v[pl.ds(k, L)]
        valid = (k + lanes) < PER_W
        r = base + k + lanes
        p = ((r >> 3) * 2000) + ((t >> 7) << 3) + (r & 7)
        idx_v[pl.ds(k, L)] = jnp.where(valid, p, k + lanes)

    pltpu.async_copy(table_hbm.at[idx_v], buf_v, sem).wait()

    acc = jnp.zeros((L,), jnp.float32)
    for j in range(CHUNKS):
        k = j * L
        t = tgt_v[pl.ds(k, L)]
        valid = (k + lanes) < PER_W
        rows = k + lanes
        lane = jnp.where(valid, t & 127, 0)
        vals = plsc.load_gather(buf_v, [rows, lane])
        acc = acc - jnp.where(valid & (t != PAD), vals, 0.0)

    red_v[...] = acc
    pltpu.sync_copy(red_v, out_hbm.at[pl.ds(wid * L, L)])


def kernel(outputs, targets):
    table = outputs.reshape(200, 8, 250, 128).transpose(0, 2, 1, 3)
    table = table.reshape(N * V // 128, 128)
    tgt = jnp.ravel(targets).astype(jnp.int32)
    tgt = jnp.pad(tgt.reshape(NW, PER_W),
                  ((0, 0), (0, SLOT - PER_W))).reshape(-1)

    mesh = plsc.VectorSubcoreMesh(
        core_axis_name="c", subcore_axis_name="s",
        num_cores=NC, num_subcores=NS)
    run = pl.kernel(
        _loss_body,
        out_type=jax.ShapeDtypeStruct((NW * L,), jnp.float32),
        mesh=mesh,
        compiler_params=pltpu.CompilerParams(use_tc_tiling_on_sc=True,
                                             needs_layout_passes=False),
        scratch_types=[
            pltpu.VMEM((SLOT,), jnp.int32),
            pltpu.VMEM((SLOT,), jnp.int32),
            pltpu.VMEM((L,), jnp.float32),
            pltpu.VMEM((SLOT, 128), jnp.float32),
            pltpu.SemaphoreType.DMA,
        ],
    )
    out = run(table, tgt)
    return jnp.sum(out)
